# Initial kernel scaffold; baseline (speedup 1.0000x reference)
import jax
import jax.numpy as jnp
from jax.experimental import pallas as pl


def kernel(x, edge_index, edge_weight, W1, b1, W2, b2):
    raise NotImplementedError("write your pallas kernel here")


# speedup vs baseline: 4.6664x; 4.6664x over previous
import functools

import jax
import jax.numpy as jnp
from jax import lax
from jax.experimental import pallas as pl
from jax.experimental.pallas import tpu as pltpu
from jax.experimental.pallas import tpu_sc as plsc

N_NODES = 10000
N_EDGES = 320000
D = 128

NC = 2
NS = 16
L = 16
NW = NC * NS
EPW = N_EDGES // NW
BLK = 80
NBLK = EPW // BLK
RCH = N_NODES // BLK

_mesh = plsc.VectorSubcoreMesh(core_axis_name="c", subcore_axis_name="s")

_sc_params = pltpu.CompilerParams()
if "needs_layout_passes" in pltpu.CompilerParams.__dataclass_fields__:
    import dataclasses as _dc
    _sc_params = _dc.replace(_sc_params, needs_layout_passes=False)


def _zero_buf(buf):
    @pl.loop(0, BLK)
    def _(e):
        for cc in range(D // L):
            buf[e, pl.ds(cc * L, L)] = jnp.zeros((L,), jnp.float32)


def _sc_agg_body(x_hbm, src_hbm, dst_hbm, w_hbm, out_hbm,
                 srcv, dstv, wv, buf, acc):
    c = lax.axis_index("c")
    s = lax.axis_index("s")
    wid = s * NC + c

    pltpu.sync_copy(src_hbm.at[wid], srcv)
    pltpu.sync_copy(dst_hbm.at[wid], dstv)

    _zero_buf(buf)
    for j in range((RCH + NS - 1) // NS):
        ch = s + NS * j

        @pl.when(ch < RCH)
        def _():
            pltpu.sync_copy(buf, acc.at[pl.ds(ch * BLK, BLK)])

    plsc.subcore_barrier()

    @pl.loop(0, NBLK)
    def _(k):
        pltpu.sync_copy(w_hbm.at[wid, k], wv)
        pltpu.sync_copy(x_hbm.at[dstv.at[k]], buf)

        @pl.loop(0, BLK)
        def _(e):
            we = plsc.load_gather(wv, [jnp.full((L,), e, jnp.int32)])
            for cc in range(D // L):
                sl = (e, pl.ds(cc * L, L))
                buf[sl] = buf[sl] * we

        pltpu.sync_copy(buf, acc.at[srcv.at[k]], add=True)

    plsc.subcore_barrier()
    for j in range((RCH + NS - 1) // NS):
        ch = s + NS * j

        @pl.when(ch < RCH)
        def _():
            pltpu.sync_copy(acc.at[pl.ds(ch * BLK, BLK)],
                            out_hbm.at[c].at[pl.ds(ch * BLK, BLK)])


_sc_agg = pl.kernel(
    _sc_agg_body,
    out_type=jax.ShapeDtypeStruct((NC, N_NODES, D), jnp.float32),
    mesh=_mesh,
    scratch_types=[
        pltpu.VMEM((NBLK, BLK), jnp.int32),
        pltpu.VMEM((NBLK, BLK), jnp.int32),
        pltpu.VMEM((BLK,), jnp.float32),
        pltpu.VMEM((BLK, D), jnp.float32),
        pltpu.VMEM_SHARED((N_NODES, D), jnp.float32),
    ],
    compiler_params=_sc_params,
)


def _sc_wsum_body(src_hbm, w_hbm, out_hbm, srcv, wv, accw):
    c = lax.axis_index("c")
    s = lax.axis_index("s")
    wid = s * NC + c
    pltpu.sync_copy(src_hbm.at[wid], srcv)
    pltpu.sync_copy(w_hbm.at[wid], wv)

    @pl.loop(0, N_NODES // L)
    def _(i):
        accw[pl.ds(i * L, L)] = jnp.zeros((L,), jnp.float32)

    @pl.loop(0, EPW // L)
    def _(i):
        idx = srcv[pl.ds(i * L, L)]
        wvv = wv[pl.ds(i * L, L)]
        plsc.addupdate_scatter(accw, [idx], wvv)

    pltpu.sync_copy(accw, out_hbm.at[wid])


_sc_wsum = pl.kernel(
    _sc_wsum_body,
    out_type=jax.ShapeDtypeStruct((NW, N_NODES), jnp.float32),
    mesh=_mesh,
    scratch_types=[
        pltpu.VMEM((EPW,), jnp.int32),
        pltpu.VMEM((EPW,), jnp.float32),
        pltpu.VMEM((N_NODES,), jnp.float32),
    ],
    compiler_params=_sc_params,
)


BR = 1000


def _tc_layer1_body(x_ref, p0_ref, p1_ref, wp_ref, wx_ref, wn_ref, b_ref,
                    h_ref, ws_ref):
    ws = jnp.clip(jnp.sum(wp_ref[0], axis=0), 1e-12, None)
    neigh = (p0_ref[...] + p1_ref[...]) / ws[:, None]
    h = jnp.dot(x_ref[...], wx_ref[...], preferred_element_type=jnp.float32)
    h = h + jnp.dot(neigh, wn_ref[...], preferred_element_type=jnp.float32)
    h = h + b_ref[...]
    h_ref[...] = jnp.maximum(h, 0.0)
    ws_ref[...] = ws[None, None, :]


def _tc_layer2_body(x_ref, p0_ref, p1_ref, ws_ref, wx_ref, wn_ref, b_ref,
                    o_ref):
    ws = ws_ref[0, 0]
    neigh = (p0_ref[...] + p1_ref[...]) / ws[:, None]
    h = jnp.dot(x_ref[...], wx_ref[...], preferred_element_type=jnp.float32)
    h = h + jnp.dot(neigh, wn_ref[...], preferred_element_type=jnp.float32)
    h = h + b_ref[...]
    h = jnp.maximum(h, 0.0)
    nrm = jnp.sqrt(jnp.sum(h * h, axis=1, keepdims=True))
    o_ref[...] = h / jnp.clip(nrm, 1e-12, None)


NBR = N_NODES // BR

_row_spec = pl.BlockSpec((BR, D), lambda i: (i, 0))
_full_w = pl.BlockSpec((D, D), lambda i: (0, 0))
_bias_spec = pl.BlockSpec((1, D), lambda i: (0, 0))
_ws_spec = pl.BlockSpec((1, 1, BR), lambda i: (i, 0, 0))

_tc_layer1 = pl.pallas_call(
    _tc_layer1_body,
    grid=(NBR,),
    in_specs=[_row_spec, _row_spec, _row_spec,
              pl.BlockSpec((1, NW, BR), lambda i: (i, 0, 0)),
              _full_w, _full_w, _bias_spec],
    out_specs=[_row_spec, _ws_spec],
    out_shape=[jax.ShapeDtypeStruct((N_NODES, D), jnp.float32),
               jax.ShapeDtypeStruct((NBR, 1, N_NODES // NBR), jnp.float32)],
)

_tc_layer2 = pl.pallas_call(
    _tc_layer2_body,
    grid=(N_NODES // BR,),
    in_specs=[_row_spec, _row_spec, _row_spec, _ws_spec,
              _full_w, _full_w, _bias_spec],
    out_specs=_row_spec,
    out_shape=jax.ShapeDtypeStruct((N_NODES, D), jnp.float32),
)


def kernel(x, edge_index, edge_weight, W1, b1, W2, b2):
    src = edge_index[0].astype(jnp.int32)
    dst = edge_index[1].astype(jnp.int32)
    src_b = src.reshape(NW, NBLK, BLK)
    dst_b = dst.reshape(NW, NBLK, BLK)
    src_f = src.reshape(NW, EPW)
    w_f = edge_weight.astype(jnp.float32).reshape(NW, EPW)

    w1x = W1[:, :D].T
    w1n = W1[:, D:].T
    w2x = W2[:, :D].T
    w2n = W2[:, D:].T
    b1r = b1.reshape(1, D)
    b2r = b2.reshape(1, D)

    w_b = edge_weight.astype(jnp.float32).reshape(NW, NBLK, BLK)

    wpart = _sc_wsum(src_f, w_f)
    wpart = wpart.reshape(NW, NBR, BR).transpose(1, 0, 2)
    p = _sc_agg(x, src_b, dst_b, w_b)
    h1, ws = _tc_layer1(x, p[0], p[1], wpart, w1x, w1n, b1r)
    q = _sc_agg(h1, src_b, dst_b, w_b)
    out = _tc_layer2(h1, q[0], q[1], ws, w2x, w2n, b2r)
    return out


# speedup vs baseline: 9.4747x; 2.0304x over previous
import functools

import jax
import jax.numpy as jnp
from jax import lax
from jax.experimental import pallas as pl
from jax.experimental.pallas import tpu as pltpu
from jax.experimental.pallas import tpu_sc as plsc

N_NODES = 10000
N_EDGES = 320000
D = 128

NC = 2
NS = 16
L = 16
NW = NC * NS
EPW = N_EDGES // NW
BLK = 80
NBLK = EPW // BLK
RCH = N_NODES // BLK

_mesh = plsc.VectorSubcoreMesh(core_axis_name="c", subcore_axis_name="s")

_sc_params = pltpu.CompilerParams()
if "needs_layout_passes" in pltpu.CompilerParams.__dataclass_fields__:
    import dataclasses as _dc
    _sc_params = _dc.replace(_sc_params, needs_layout_passes=False)


def _zero_buf(buf):
    @pl.loop(0, BLK)
    def _(e):
        for cc in range(D // L):
            buf[e, pl.ds(cc * L, L)] = jnp.zeros((L,), jnp.float32)


def _sc_agg_body(x_hbm, pk_hbm, w_hbm, out_hbm,
                 pkv, buf0, buf1, sb0, db0, sb1, db1, wv0, wv1,
                 gs0, gs1, ws0, ws1, ss0, ss1, acc):
    c = lax.axis_index("c")
    s = lax.axis_index("s")
    wid = s * NC + c

    def unpack(k, sb, db):
        for j in range(BLK // L):
            sl = pl.ds(j * L, L)
            p = pkv[k, sl]
            sb[sl] = lax.shift_right_logical(p, 16)
            db[sl] = lax.bitwise_and(p, 0xFFFF)

    def issue_w(k, wv, sem):
        pltpu.async_copy(w_hbm.at[wid, k], wv, sem)

    def wait_w(wv, sem):
        pltpu.make_async_copy(w_hbm.at[0, 0], wv, sem).wait()

    def issue_gather(db, buf, sem):
        pltpu.async_copy(x_hbm.at[db], buf, sem)

    def wait_gather(buf, sem):
        pltpu.make_async_copy(x_hbm.at[pl.ds(0, BLK)], buf, sem).wait()

    def scale(buf, wv):
        @plsc.parallel_loop(0, BLK, unroll=4)
        def _(e):
            we = plsc.load_gather(wv, [jnp.full((L,), e, jnp.int32)])
            for cc in range(D // L):
                sl = (e, pl.ds(cc * L, L))
                buf[sl] = buf[sl] * we

    def issue_scatter(buf, sb, sem):
        pltpu.async_copy(buf, acc.at[sb], sem, add=True)

    def wait_scatter(buf, sb, sem):
        pltpu.make_async_copy(buf, acc.at[sb], sem).wait()

    pltpu.sync_copy(pk_hbm.at[wid], pkv)

    _zero_buf(buf0)
    for j in range((RCH + NS - 1) // NS):
        ch = s + NS * j

        @pl.when(ch < RCH)
        def _():
            pltpu.sync_copy(buf0, acc.at[pl.ds(ch * BLK, BLK)])

    plsc.subcore_barrier()

    unpack(0, sb0, db0)
    issue_w(0, wv0, ws0)
    issue_gather(db0, buf0, gs0)

    @pl.loop(0, NBLK - 1, step=2)
    def _(k):
        @pl.when(k > 0)
        def _():
            wait_scatter(buf1, sb1, ss1)

        unpack(k + 1, sb1, db1)
        issue_w(k + 1, wv1, ws1)
        issue_gather(db1, buf1, gs1)

        wait_gather(buf0, gs0)
        wait_w(wv0, ws0)
        scale(buf0, wv0)
        issue_scatter(buf0, sb0, ss0)

        wait_gather(buf1, gs1)
        wait_w(wv1, ws1)
        scale(buf1, wv1)
        issue_scatter(buf1, sb1, ss1)

        wait_scatter(buf0, sb0, ss0)
        unpack(k + 2, sb0, db0)
        issue_w(k + 2, wv0, ws0)
        issue_gather(db0, buf0, gs0)

    wait_gather(buf0, gs0)
    wait_w(wv0, ws0)
    scale(buf0, wv0)
    issue_scatter(buf0, sb0, ss0)
    wait_scatter(buf1, sb1, ss1)
    wait_scatter(buf0, sb0, ss0)

    plsc.subcore_barrier()
    for j in range((RCH + NS - 1) // NS):
        ch = s + NS * j

        @pl.when(ch < RCH)
        def _():
            pltpu.sync_copy(acc.at[pl.ds(ch * BLK, BLK)],
                            out_hbm.at[c].at[pl.ds(ch * BLK, BLK)])


_sc_agg = pl.kernel(
    _sc_agg_body,
    out_type=jax.ShapeDtypeStruct((NC, N_NODES, D), jnp.float32),
    mesh=_mesh,
    scratch_types=[
        pltpu.VMEM((NBLK, BLK), jnp.int32),
        pltpu.VMEM((BLK, D), jnp.float32),
        pltpu.VMEM((BLK, D), jnp.float32),
        pltpu.VMEM((BLK,), jnp.int32),
        pltpu.VMEM((BLK,), jnp.int32),
        pltpu.VMEM((BLK,), jnp.int32),
        pltpu.VMEM((BLK,), jnp.int32),
        pltpu.VMEM((BLK,), jnp.float32),
        pltpu.VMEM((BLK,), jnp.float32),
        pltpu.SemaphoreType.DMA,
        pltpu.SemaphoreType.DMA,
        pltpu.SemaphoreType.DMA,
        pltpu.SemaphoreType.DMA,
        pltpu.SemaphoreType.DMA,
        pltpu.SemaphoreType.DMA,
        pltpu.VMEM_SHARED((N_NODES, D), jnp.float32),
    ],
    compiler_params=_sc_params,
)


def _sc_wsum_body(src_hbm, w_hbm, out_hbm, srcv, wv, accw):
    c = lax.axis_index("c")
    s = lax.axis_index("s")
    wid = s * NC + c
    pltpu.sync_copy(src_hbm.at[wid], srcv)
    pltpu.sync_copy(w_hbm.at[wid], wv)

    @pl.loop(0, N_NODES // L)
    def _(i):
        accw[pl.ds(i * L, L)] = jnp.zeros((L,), jnp.float32)

    @pl.loop(0, EPW // L)
    def _(i):
        idx = srcv[pl.ds(i * L, L)]
        wvv = wv[pl.ds(i * L, L)]
        plsc.addupdate_scatter(accw, [idx], wvv)

    pltpu.sync_copy(accw, out_hbm.at[wid])


_sc_wsum = pl.kernel(
    _sc_wsum_body,
    out_type=jax.ShapeDtypeStruct((NW, N_NODES), jnp.float32),
    mesh=_mesh,
    scratch_types=[
        pltpu.VMEM((EPW,), jnp.int32),
        pltpu.VMEM((EPW,), jnp.float32),
        pltpu.VMEM((N_NODES,), jnp.float32),
    ],
    compiler_params=_sc_params,
)


BR = 1000


def _tc_layer1_body(x_ref, p0_ref, p1_ref, wp_ref, wx_ref, wn_ref, b_ref,
                    h_ref, ws_ref):
    ws = jnp.clip(jnp.sum(wp_ref[0], axis=0), 1e-12, None)
    neigh = (p0_ref[...] + p1_ref[...]) / ws[:, None]
    h = jnp.dot(x_ref[...], wx_ref[...], preferred_element_type=jnp.float32)
    h = h + jnp.dot(neigh, wn_ref[...], preferred_element_type=jnp.float32)
    h = h + b_ref[...]
    h_ref[...] = jnp.maximum(h, 0.0)
    ws_ref[...] = ws[None, None, :]


def _tc_layer2_body(x_ref, p0_ref, p1_ref, ws_ref, wx_ref, wn_ref, b_ref,
                    o_ref):
    ws = ws_ref[0, 0]
    neigh = (p0_ref[...] + p1_ref[...]) / ws[:, None]
    h = jnp.dot(x_ref[...], wx_ref[...], preferred_element_type=jnp.float32)
    h = h + jnp.dot(neigh, wn_ref[...], preferred_element_type=jnp.float32)
    h = h + b_ref[...]
    h = jnp.maximum(h, 0.0)
    nrm = jnp.sqrt(jnp.sum(h * h, axis=1, keepdims=True))
    o_ref[...] = h / jnp.clip(nrm, 1e-12, None)


NBR = N_NODES // BR

_row_spec = pl.BlockSpec((BR, D), lambda i: (i, 0))
_full_w = pl.BlockSpec((D, D), lambda i: (0, 0))
_bias_spec = pl.BlockSpec((1, D), lambda i: (0, 0))
_ws_spec = pl.BlockSpec((1, 1, BR), lambda i: (i, 0, 0))

_tc_layer1 = pl.pallas_call(
    _tc_layer1_body,
    grid=(NBR,),
    in_specs=[_row_spec, _row_spec, _row_spec,
              pl.BlockSpec((1, NW, BR), lambda i: (i, 0, 0)),
              _full_w, _full_w, _bias_spec],
    out_specs=[_row_spec, _ws_spec],
    out_shape=[jax.ShapeDtypeStruct((N_NODES, D), jnp.float32),
               jax.ShapeDtypeStruct((NBR, 1, N_NODES // NBR), jnp.float32)],
)

_tc_layer2 = pl.pallas_call(
    _tc_layer2_body,
    grid=(N_NODES // BR,),
    in_specs=[_row_spec, _row_spec, _row_spec, _ws_spec,
              _full_w, _full_w, _bias_spec],
    out_specs=_row_spec,
    out_shape=jax.ShapeDtypeStruct((N_NODES, D), jnp.float32),
)


def kernel(x, edge_index, edge_weight, W1, b1, W2, b2):
    src = edge_index[0].astype(jnp.int32)
    dst = edge_index[1].astype(jnp.int32)
    packed = ((src << 16) | dst).reshape(NW, NBLK, BLK)
    src_f = src.reshape(NW, EPW)
    w_f = edge_weight.astype(jnp.float32).reshape(NW, EPW)

    w1x = W1[:, :D].T
    w1n = W1[:, D:].T
    w2x = W2[:, :D].T
    w2n = W2[:, D:].T
    b1r = b1.reshape(1, D)
    b2r = b2.reshape(1, D)

    w_b = edge_weight.astype(jnp.float32).reshape(NW, NBLK, BLK)

    wpart = _sc_wsum(src_f, w_f)
    wpart = wpart.reshape(NW, NBR, BR).transpose(1, 0, 2)
    p = _sc_agg(x, packed, w_b)
    h1, ws = _tc_layer1(x, p[0], p[1], wpart, w1x, w1n, b1r)
    q = _sc_agg(h1, packed, w_b)
    out = _tc_layer2(h1, q[0], q[1], ws, w2x, w2n, b2r)
    return out
